# baseline (device time: 83879 ns/iter reference)
import jax
import jax.numpy as jnp
from jax import lax
from jax.experimental import pallas as pl
from jax.experimental.pallas import tpu as pltpu

N_DEV = 8
NCHUNK = 4
NW = 6
SEND_WIN = 3


def kernel(x, w_mat):
    M, k_per = x.shape
    K, N = w_mat.shape
    m_per = M // N_DEV

    def body(x_hbm, w_hbm, out_ref, xv, x16, comm_ref, w_buf, send_sems,
             recv_sems, w_sems, x_sem):
        me = lax.axis_index("i")
        b = lax.div(me, 4) * 4
        s = me - b
        fb = 4 - b

        barrier = pltpu.get_barrier_semaphore()
        for off in range(1, N_DEV):
            peer = lax.rem(me + off, N_DEV)
            pl.semaphore_signal(
                barrier, inc=1,
                device_id=(peer,), device_id_type=pl.DeviceIdType.MESH,
            )

        xdma = pltpu.make_async_copy(x_hbm, xv, x_sem)
        xdma.start()

        def consume_dev(h):
            if h == 0:
                return me
            if h <= 3:
                return b + lax.rem(s - h + 4, 4)
            return fb + lax.rem(s - (h - 4) + 4, 4)

        n_chunk = N // NCHUNK
        n_steps = NCHUNK * N_DEV

        def wdma(t, slot):
            h, c = divmod(t, NCHUNK)
            return pltpu.make_async_copy(
                w_hbm.at[pl.ds(consume_dev(h) * k_per, k_per),
                         pl.ds(c * n_chunk, n_chunk)],
                w_buf.at[slot],
                w_sems.at[slot],
            )

        w_descs = {}
        for t in range(NW - 1):
            w_descs[t % NW] = wdma(t, t % NW)
            w_descs[t % NW].start()

        xdma.wait()
        x16[:, :] = xv[:, :].astype(jnp.bfloat16)
        pl.semaphore_wait(barrier, N_DEV - 1)

        sends = []
        dsts = [b + lax.rem(s + i, 4) for i in range(1, 4)]
        dsts += [fb + lax.rem(s + k, 4) for k in range(4)]
        for idx, dst in enumerate(dsts):
            rdma = pltpu.make_async_remote_copy(
                src_ref=x16.at[pl.ds(dst * m_per, m_per), :],
                dst_ref=comm_ref.at[me],
                send_sem=send_sems.at[idx],
                recv_sem=recv_sems.at[me],
                device_id=(dst,),
                device_id_type=pl.DeviceIdType.MESH,
            )
            sends.append(rdma)
        for idx in range(SEND_WIN):
            sends[idx].start()

        for t in range(n_steps):
            h, c = divmod(t, NCHUNK)
            j = consume_dev(h)
            slot = t % NW
            if t + NW - 1 < n_steps:
                nxt_slot = (t + NW - 1) % NW
                w_descs[nxt_slot] = wdma(t + NW - 1, nxt_slot)
                w_descs[nxt_slot].start()
            if h > 0 and c == 0:
                recv = pltpu.make_async_remote_copy(
                    src_ref=comm_ref.at[j],
                    dst_ref=comm_ref.at[j],
                    send_sem=send_sems.at[0],
                    recv_sem=recv_sems.at[j],
                    device_id=(j,),
                    device_id_type=pl.DeviceIdType.MESH,
                )
                recv.wait_recv()
                if h + SEND_WIN < N_DEV:
                    sends[h + SEND_WIN - 1].start()
            w_descs[slot].wait()
            xblk = (
                xv[pl.ds(me * m_per, m_per), :]
                if h == 0
                else comm_ref[j].astype(jnp.float32)
            )
            partial = jnp.dot(
                xblk, w_buf[slot], preferred_element_type=jnp.float32
            )
            if h == 0:
                out_ref[:, pl.ds(c * n_chunk, n_chunk)] = partial
            else:
                out_ref[:, pl.ds(c * n_chunk, n_chunk)] += partial

        for snd in sends:
            snd.wait_send()

    return pl.pallas_call(
        body,
        out_shape=jax.ShapeDtypeStruct((m_per, N), jnp.float32),
        in_specs=[
            pl.BlockSpec(memory_space=pl.ANY),
            pl.BlockSpec(memory_space=pl.ANY),
        ],
        out_specs=pl.BlockSpec(memory_space=pltpu.VMEM),
        scratch_shapes=[
            pltpu.VMEM((M, k_per), jnp.float32),
            pltpu.VMEM((M, k_per), jnp.bfloat16),
            pltpu.VMEM((N_DEV, m_per, k_per), jnp.bfloat16),
            pltpu.VMEM((NW, k_per, N // NCHUNK), jnp.float32),
            pltpu.SemaphoreType.DMA((N_DEV - 1,)),
            pltpu.SemaphoreType.DMA((N_DEV,)),
            pltpu.SemaphoreType.DMA((NW,)),
            pltpu.SemaphoreType.DMA,
        ],
        compiler_params=pltpu.CompilerParams(
            collective_id=0,
            vmem_limit_bytes=100 * 1024 * 1024,
        ),
    )(x, w_mat)


# device time: 79551 ns/iter; 1.0544x vs baseline; 1.0544x over previous
import jax
import jax.numpy as jnp
from jax import lax
from jax.experimental import pallas as pl
from jax.experimental.pallas import tpu as pltpu

N_DEV = 8
NCHUNK = 4
NW = 6


def kernel(x, w_mat):
    M, k_per = x.shape
    K, N = w_mat.shape
    m_per = M // N_DEV

    def body(x_ref, w_hbm, out_ref, x16, comm_ref, w_buf, send_sems,
             recv_sems, w_sems):
        me = lax.axis_index("i")
        b = lax.div(me, 4) * 4
        s = me - b
        fb = 4 - b

        barrier = pltpu.get_barrier_semaphore()
        for off in range(1, N_DEV):
            peer = lax.rem(me + off, N_DEV)
            pl.semaphore_signal(
                barrier, inc=1,
                device_id=(peer,), device_id_type=pl.DeviceIdType.MESH,
            )
        x16[:, :] = x_ref[:, :].astype(jnp.bfloat16)
        pl.semaphore_wait(barrier, N_DEV - 1)

        sends = []
        dsts = [b + lax.rem(s + i, 4) for i in range(1, 4)]
        dsts += [fb + lax.rem(s + k, 4) for k in range(4)]
        for idx, dst in enumerate(dsts):
            rdma = pltpu.make_async_remote_copy(
                src_ref=x16.at[pl.ds(dst * m_per, m_per), :],
                dst_ref=comm_ref.at[me],
                send_sem=send_sems.at[idx],
                recv_sem=recv_sems.at[me],
                device_id=(dst,),
                device_id_type=pl.DeviceIdType.MESH,
            )
            rdma.start()
            sends.append(rdma)

        def consume_dev(h):
            if h == 0:
                return me
            if h <= 3:
                return b + lax.rem(s - h + 4, 4)
            return fb + lax.rem(s - (h - 4) + 4, 4)

        n_chunk = N // NCHUNK
        n_steps = NCHUNK * N_DEV

        def wdma(t, slot):
            h, c = divmod(t, NCHUNK)
            return pltpu.make_async_copy(
                w_hbm.at[pl.ds(consume_dev(h) * k_per, k_per),
                         pl.ds(c * n_chunk, n_chunk)],
                w_buf.at[slot],
                w_sems.at[slot],
            )

        w_descs = {}
        for t in range(NW - 1):
            w_descs[t % NW] = wdma(t, t % NW)
            w_descs[t % NW].start()

        for t in range(n_steps):
            h, c = divmod(t, NCHUNK)
            j = consume_dev(h)
            slot = t % NW
            if t + NW - 1 < n_steps:
                nxt_slot = (t + NW - 1) % NW
                w_descs[nxt_slot] = wdma(t + NW - 1, nxt_slot)
                w_descs[nxt_slot].start()
            if h > 0 and c == 0:
                recv = pltpu.make_async_remote_copy(
                    src_ref=comm_ref.at[j],
                    dst_ref=comm_ref.at[j],
                    send_sem=send_sems.at[0],
                    recv_sem=recv_sems.at[j],
                    device_id=(j,),
                    device_id_type=pl.DeviceIdType.MESH,
                )
                recv.wait_recv()
            w_descs[slot].wait()
            xblk = (
                x_ref[pl.ds(me * m_per, m_per), :]
                if h == 0
                else comm_ref[j].astype(jnp.float32)
            )
            partial = jnp.dot(
                xblk, w_buf[slot], preferred_element_type=jnp.float32
            )
            if h == 0:
                out_ref[:, pl.ds(c * n_chunk, n_chunk)] = partial
            else:
                out_ref[:, pl.ds(c * n_chunk, n_chunk)] += partial

        for snd in sends:
            snd.wait_send()

    return pl.pallas_call(
        body,
        out_shape=jax.ShapeDtypeStruct((m_per, N), jnp.float32),
        in_specs=[
            pl.BlockSpec(memory_space=pltpu.VMEM),
            pl.BlockSpec(memory_space=pl.ANY),
        ],
        out_specs=pl.BlockSpec(memory_space=pltpu.VMEM),
        scratch_shapes=[
            pltpu.VMEM((M, k_per), jnp.bfloat16),
            pltpu.VMEM((N_DEV, m_per, k_per), jnp.bfloat16),
            pltpu.VMEM((NW, k_per, N // NCHUNK), jnp.float32),
            pltpu.SemaphoreType.DMA((N_DEV - 1,)),
            pltpu.SemaphoreType.DMA((N_DEV,)),
            pltpu.SemaphoreType.DMA((NW,)),
        ],
        compiler_params=pltpu.CompilerParams(
            collective_id=0,
            vmem_limit_bytes=100 * 1024 * 1024,
        ),
    )(x, w_mat)


# device time: 77348 ns/iter; 1.0844x vs baseline; 1.0285x over previous
import jax
import jax.numpy as jnp
from jax import lax
from jax.experimental import pallas as pl
from jax.experimental.pallas import tpu as pltpu

N_DEV = 8
NCHUNK = 4
NW = 6


def kernel(x, w_mat):
    M, k_per = x.shape
    K, N = w_mat.shape
    m_per = M // N_DEV

    def body(x_ref, w_hbm, out_ref, x16, comm_ref, w_buf, send_sems,
             recv_sems, w_sems):
        me = lax.axis_index("i")
        b = lax.div(me, 4) * 4
        s = me - b
        fb = 4 - b

        barrier = pltpu.get_barrier_semaphore()
        for off in range(1, N_DEV):
            peer = lax.rem(me + off, N_DEV)
            pl.semaphore_signal(
                barrier, inc=1,
                device_id=(peer,), device_id_type=pl.DeviceIdType.MESH,
            )
        def consume_dev(h):
            if h == 0:
                return me
            if h <= 3:
                return b + lax.rem(s - h + 4, 4)
            return fb + lax.rem(s - (h - 4) + 4, 4)

        n_chunk = N // NCHUNK
        n_steps = NCHUNK * N_DEV

        def wdma(t, slot):
            h, c = divmod(t, NCHUNK)
            return pltpu.make_async_copy(
                w_hbm.at[pl.ds(consume_dev(h) * k_per, k_per),
                         pl.ds(c * n_chunk, n_chunk)],
                w_buf.at[slot],
                w_sems.at[slot],
            )

        w_descs = {}
        for t in range(NW - 1):
            w_descs[t % NW] = wdma(t, t % NW)
            w_descs[t % NW].start()

        x16[:, :] = x_ref[:, :].astype(jnp.bfloat16)
        pl.semaphore_wait(barrier, N_DEV - 1)

        sends = []
        dsts = [b + lax.rem(s + i, 4) for i in range(1, 4)]
        dsts += [fb + lax.rem(s + k, 4) for k in range(4)]
        for idx, dst in enumerate(dsts):
            rdma = pltpu.make_async_remote_copy(
                src_ref=x16.at[pl.ds(dst * m_per, m_per), :],
                dst_ref=comm_ref.at[me],
                send_sem=send_sems.at[idx],
                recv_sem=recv_sems.at[me],
                device_id=(dst,),
                device_id_type=pl.DeviceIdType.MESH,
            )
            rdma.start()
            sends.append(rdma)

        for t in range(n_steps):
            h, c = divmod(t, NCHUNK)
            j = consume_dev(h)
            slot = t % NW
            if t + NW - 1 < n_steps:
                nxt_slot = (t + NW - 1) % NW
                w_descs[nxt_slot] = wdma(t + NW - 1, nxt_slot)
                w_descs[nxt_slot].start()
            if h > 0 and c == 0:
                recv = pltpu.make_async_remote_copy(
                    src_ref=comm_ref.at[j],
                    dst_ref=comm_ref.at[j],
                    send_sem=send_sems.at[0],
                    recv_sem=recv_sems.at[j],
                    device_id=(j,),
                    device_id_type=pl.DeviceIdType.MESH,
                )
                recv.wait_recv()
            w_descs[slot].wait()
            xblk = (
                x_ref[pl.ds(me * m_per, m_per), :]
                if h == 0
                else comm_ref[j].astype(jnp.float32)
            )
            partial = jnp.dot(
                xblk, w_buf[slot], preferred_element_type=jnp.float32
            )
            if h == 0:
                out_ref[:, pl.ds(c * n_chunk, n_chunk)] = partial
            else:
                out_ref[:, pl.ds(c * n_chunk, n_chunk)] += partial

        for snd in sends:
            snd.wait_send()

    return pl.pallas_call(
        body,
        out_shape=jax.ShapeDtypeStruct((m_per, N), jnp.float32),
        in_specs=[
            pl.BlockSpec(memory_space=pltpu.VMEM),
            pl.BlockSpec(memory_space=pl.ANY),
        ],
        out_specs=pl.BlockSpec(memory_space=pltpu.VMEM),
        scratch_shapes=[
            pltpu.VMEM((M, k_per), jnp.bfloat16),
            pltpu.VMEM((N_DEV, m_per, k_per), jnp.bfloat16),
            pltpu.VMEM((NW, k_per, N // NCHUNK), jnp.float32),
            pltpu.SemaphoreType.DMA((N_DEV - 1,)),
            pltpu.SemaphoreType.DMA((N_DEV,)),
            pltpu.SemaphoreType.DMA((NW,)),
        ],
        compiler_params=pltpu.CompilerParams(
            collective_id=0,
            vmem_limit_bytes=100 * 1024 * 1024,
        ),
    )(x, w_mat)
